# speedup vs baseline: 1.5164x; 1.5164x over previous
import jax
import jax.numpy as jnp
from jax import lax
from jax.experimental import pallas as pl
from jax.experimental.pallas import tpu as pltpu
from jax.experimental.pallas import tpu_sc as plsc

VOCAB_SIZE = 32000
HIDDEN_SIZE = 4096
BATCH = 4
SEQ_LEN = 4096

NC = 2
NS = 16
NW = NC * NS

B = BATCH * SEQ_LEN
BPW = B // NW
K = 8
CH = BPW // K

_mesh = plsc.VectorSubcoreMesh(
    core_axis_name="c", subcore_axis_name="s", num_cores=NC, num_subcores=NS
)


@jax.jit
def _embed_gather(idx3, table):
    @pl.kernel(
        out_type=jax.ShapeDtypeStruct((B, HIDDEN_SIZE), jnp.float32),
        mesh=_mesh,
        scratch_types=[
            pltpu.VMEM((CH, K), jnp.int32),
            pltpu.VMEM((K, HIDDEN_SIZE), jnp.float32),
            pltpu.SemaphoreType.DMA,
        ],
    )
    def k(idx_hbm, table_hbm, out_hbm, idx_v, rows, gsem):
        wid = lax.axis_index("s") * NC + lax.axis_index("c")
        base = wid * BPW
        pltpu.sync_copy(idx_hbm.at[wid], idx_v)

        def chunk(j, _):
            pltpu.async_copy(table_hbm.at[idx_v.at[j]], rows, gsem).wait()
            pltpu.sync_copy(rows, out_hbm.at[pl.ds(base + j * K, K)])
            return 0

        lax.fori_loop(0, CH, chunk, 0)

    return k(idx3, table)


def kernel(input_ids, embed_weight):
    idx3 = input_ids.reshape(NW, CH, K)
    out = _embed_gather(idx3, embed_weight)
    return out.reshape(BATCH, SEQ_LEN, HIDDEN_SIZE)


# speedup vs baseline: 1.7679x; 1.1659x over previous
import jax
import jax.numpy as jnp
from jax import lax
from jax.experimental import pallas as pl
from jax.experimental.pallas import tpu as pltpu
from jax.experimental.pallas import tpu_sc as plsc

VOCAB_SIZE = 32000
HIDDEN_SIZE = 4096
BATCH = 4
SEQ_LEN = 4096

NC = 2
NS = 16
NW = NC * NS

B = BATCH * SEQ_LEN
BPW = B // NW
K = 8
CH = BPW // K

_mesh = plsc.VectorSubcoreMesh(
    core_axis_name="c", subcore_axis_name="s", num_cores=NC, num_subcores=NS
)


NBUF = 2


@jax.jit
def _embed_gather(idx3, table):
    @pl.kernel(
        out_type=jax.ShapeDtypeStruct((B, HIDDEN_SIZE), jnp.float32),
        mesh=_mesh,
        scratch_types=[
            pltpu.VMEM((CH, K), jnp.int32),
            *[pltpu.VMEM((K, HIDDEN_SIZE), jnp.float32) for _ in range(NBUF)],
            *[pltpu.SemaphoreType.DMA for _ in range(2 * NBUF)],
        ],
    )
    def k(idx_hbm, table_hbm, out_hbm, idx_v, *scr):
        bufs = scr[:NBUF]
        gsems = scr[NBUF:2 * NBUF]
        wsems = scr[2 * NBUF:]
        wid = lax.axis_index("s") * NC + lax.axis_index("c")
        base = wid * BPW
        pltpu.sync_copy(idx_hbm.at[wid], idx_v)

        for b in range(NBUF):
            pltpu.async_copy(table_hbm.at[idx_v.at[b]], bufs[b], gsems[b])

        def step(g, _):
            for b in range(NBUF):
                j = g * NBUF + b
                pltpu.make_async_copy(
                    table_hbm.at[idx_v.at[j]], bufs[b], gsems[b]
                ).wait()
                pltpu.async_copy(
                    bufs[b], out_hbm.at[pl.ds(base + j * K, K)], wsems[b]
                )
            for b in range(NBUF):
                j = g * NBUF + b
                jn = j + NBUF
                pltpu.make_async_copy(
                    bufs[b], out_hbm.at[pl.ds(base + j * K, K)], wsems[b]
                ).wait()

                @pl.when(jn < CH)
                def _():
                    pltpu.async_copy(table_hbm.at[idx_v.at[jn]], bufs[b], gsems[b])

            return 0

        lax.fori_loop(0, CH // NBUF, step, 0)

    return k(idx3, table)


def kernel(input_ids, embed_weight):
    idx3 = input_ids.reshape(NW, CH, K)
    out = _embed_gather(idx3, embed_weight)
    return out.reshape(BATCH, SEQ_LEN, HIDDEN_SIZE)


# speedup vs baseline: 1.7877x; 1.0112x over previous
import jax
import jax.numpy as jnp
from jax import lax
from jax.experimental import pallas as pl
from jax.experimental.pallas import tpu as pltpu
from jax.experimental.pallas import tpu_sc as plsc

VOCAB_SIZE = 32000
HIDDEN_SIZE = 4096
BATCH = 4
SEQ_LEN = 4096

NC = 2
NS = 16
NW = NC * NS

B = BATCH * SEQ_LEN
BPW = B // NW
K = 4
CH = BPW // K

_mesh = plsc.VectorSubcoreMesh(
    core_axis_name="c", subcore_axis_name="s", num_cores=NC, num_subcores=NS
)


NBUF = 4


@jax.jit
def _embed_gather(idx3, table):
    @pl.kernel(
        out_type=jax.ShapeDtypeStruct((B, HIDDEN_SIZE), jnp.float32),
        mesh=_mesh,
        scratch_types=[
            pltpu.VMEM((CH, K), jnp.int32),
            *[pltpu.VMEM((K, HIDDEN_SIZE), jnp.float32) for _ in range(NBUF)],
            *[pltpu.SemaphoreType.DMA for _ in range(2 * NBUF)],
        ],
    )
    def k(idx_hbm, table_hbm, out_hbm, idx_v, *scr):
        bufs = scr[:NBUF]
        gsems = scr[NBUF:2 * NBUF]
        wsems = scr[2 * NBUF:]
        wid = lax.axis_index("s") * NC + lax.axis_index("c")
        base = wid * BPW
        pltpu.sync_copy(idx_hbm.at[wid], idx_v)

        for b in range(NBUF):
            pltpu.async_copy(table_hbm.at[idx_v.at[b]], bufs[b], gsems[b])

        def step(g, _):
            for b in range(NBUF):
                j = g * NBUF + b
                pltpu.make_async_copy(
                    table_hbm.at[idx_v.at[j]], bufs[b], gsems[b]
                ).wait()
                pltpu.async_copy(
                    bufs[b], out_hbm.at[pl.ds(base + j * K, K)], wsems[b]
                )
            for b in range(NBUF):
                j = g * NBUF + b
                jn = j + NBUF
                pltpu.make_async_copy(
                    bufs[b], out_hbm.at[pl.ds(base + j * K, K)], wsems[b]
                ).wait()

                @pl.when(jn < CH)
                def _():
                    pltpu.async_copy(table_hbm.at[idx_v.at[jn]], bufs[b], gsems[b])

            return 0

        lax.fori_loop(0, CH // NBUF, step, 0)

    return k(idx3, table)


def kernel(input_ids, embed_weight):
    idx3 = input_ids.reshape(NW, CH, K)
    out = _embed_gather(idx3, embed_weight)
    return out.reshape(BATCH, SEQ_LEN, HIDDEN_SIZE)
